# baseline (speedup 1.0000x reference)
import jax
import jax.numpy as jnp
from jax import lax
from jax.experimental import pallas as pl
from jax.experimental.pallas import tpu as pltpu
from jax.experimental.pallas import tpu_sc as plsc

N_VOTERS = 32
N_COLS = 1_000_000
LANES = 16
NUM_WORKERS = 32
PER_WORKER = 17664
NBUF = 3
CHUNK = 1280
CHUNK_SIZES = [384] + [CHUNK] * 13 + [640]
assert sum(CHUNK_SIZES) == PER_WORKER and all(s % 128 == 0 for s in CHUNK_SIZES)
NCHUNKS = len(CHUNK_SIZES)
CHUNK_OFFS = [sum(CHUNK_SIZES[:i]) for i in range(NCHUNKS)]
SC_COLS = NUM_WORKERS * PER_WORKER
TC_BLOCK = 8192
TC_COLS = N_COLS - SC_COLS
TC_BLOCKS = -(-TC_COLS // TC_BLOCK)


def _body(
    in_hbm,
    out_hbm,
    buf0,
    buf1,
    buf2,
    obuf0,
    obuf1,
    isem0,
    isem1,
    isem2,
    osem0,
    osem1,
):
    c = lax.axis_index("c")
    s = lax.axis_index("s")
    wid = s * 2 + c
    base = wid * PER_WORKER
    bufs = (buf0, buf1, buf2)
    obufs = (obuf0, obuf1)
    isems = (isem0, isem1, isem2)
    osems = (osem0, osem1)

    def in_copy(k):
        return pltpu.make_async_copy(
            in_hbm.at[:, pl.ds(base + CHUNK_OFFS[k], CHUNK_SIZES[k])],
            bufs[k % NBUF].at[:, pl.ds(0, CHUNK_SIZES[k])],
            isems[k % NBUF],
        )

    def out_copy(k):
        return pltpu.make_async_copy(
            obufs[k % 2].at[pl.ds(0, CHUNK_SIZES[k])],
            out_hbm.at[pl.ds(base + CHUNK_OFFS[k], CHUNK_SIZES[k])],
            osems[k % 2],
        )

    def reduce_cols(src, dst, n_cols):
        @plsc.parallel_loop(0, n_cols // LANES, unroll=1)
        def col_group(j):
            off = j * LANES
            vals = [src[i, pl.ds(off, LANES)] for i in range(N_VOTERS)]
            while len(vals) > 1:
                vals = [
                    vals[i] + vals[i + 1] for i in range(0, len(vals), 2)
                ]
            dst[pl.ds(off, LANES)] = jnp.where(
                vals[0] > N_VOTERS // 2, 1, 0
            ).astype(jnp.int32)

    for k in range(NBUF):
        in_copy(k).start()
    for k in range(NCHUNKS):
        in_copy(k).wait()
        if k >= 2:
            out_copy(k - 2).wait()
        reduce_cols(bufs[k % NBUF], obufs[k % 2], CHUNK_SIZES[k])
        out_copy(k).start()
        if k + NBUF < NCHUNKS:
            in_copy(k + NBUF).start()
    out_copy(NCHUNKS - 2).wait()
    out_copy(NCHUNKS - 1).wait()


def _tc_body(x_ref, o_ref):
    o_ref[...] = (
        jnp.sum(x_ref[...], axis=0) > N_VOTERS // 2
    ).astype(jnp.int32)


@jax.jit
def _vote(inputs):
    out_tc = pl.pallas_call(
        _tc_body,
        grid=(TC_BLOCKS,),
        in_specs=[
            pl.BlockSpec(
                (N_VOTERS, TC_BLOCK), lambda j: (0, j + SC_COLS // TC_BLOCK)
            )
        ],
        out_specs=pl.BlockSpec((TC_BLOCK,), lambda j: (j,)),
        out_shape=jax.ShapeDtypeStruct((TC_COLS,), jnp.int32),
    )(inputs)
    k = pl.kernel(
        _body,
        out_type=jax.ShapeDtypeStruct((SC_COLS,), jnp.int32),
        mesh=plsc.VectorSubcoreMesh(core_axis_name="c", subcore_axis_name="s"),
        scratch_types=[
            pltpu.VMEM((N_VOTERS, CHUNK), jnp.int32),
            pltpu.VMEM((N_VOTERS, CHUNK), jnp.int32),
            pltpu.VMEM((N_VOTERS, CHUNK), jnp.int32),
            pltpu.VMEM((CHUNK,), jnp.int32),
            pltpu.VMEM((CHUNK,), jnp.int32),
            pltpu.SemaphoreType.DMA,
            pltpu.SemaphoreType.DMA,
            pltpu.SemaphoreType.DMA,
            pltpu.SemaphoreType.DMA,
            pltpu.SemaphoreType.DMA,
        ],
    )
    out_sc = k(inputs)
    return jnp.concatenate([out_sc, out_tc])


def kernel(inputs):
    return _vote(inputs)


# speedup vs baseline: 1.0063x; 1.0063x over previous
import jax
import jax.numpy as jnp
from jax import lax
from jax.experimental import pallas as pl
from jax.experimental.pallas import tpu as pltpu
from jax.experimental.pallas import tpu_sc as plsc

N_VOTERS = 32
N_COLS = 1_000_000
LANES = 16
NUM_WORKERS = 32
PER_WORKER = 20480
NBUF = 3
CHUNK = 1280
CHUNK_SIZES = [384] + [CHUNK] * 15 + [896]
assert sum(CHUNK_SIZES) == PER_WORKER and all(s % 128 == 0 for s in CHUNK_SIZES)
NCHUNKS = len(CHUNK_SIZES)
CHUNK_OFFS = [sum(CHUNK_SIZES[:i]) for i in range(NCHUNKS)]
SC_COLS = NUM_WORKERS * PER_WORKER
TC_BLOCK = 8192
TC_COLS = N_COLS - SC_COLS
TC_BLOCKS = -(-TC_COLS // TC_BLOCK)


def _body(
    in_hbm,
    out_hbm,
    buf0,
    buf1,
    buf2,
    obuf0,
    obuf1,
    isem0,
    isem1,
    isem2,
    osem0,
    osem1,
):
    c = lax.axis_index("c")
    s = lax.axis_index("s")
    wid = s * 2 + c
    base = wid * PER_WORKER
    bufs = (buf0, buf1, buf2)
    obufs = (obuf0, obuf1)
    isems = (isem0, isem1, isem2)
    osems = (osem0, osem1)

    def in_copy(k):
        return pltpu.make_async_copy(
            in_hbm.at[:, pl.ds(base + CHUNK_OFFS[k], CHUNK_SIZES[k])],
            bufs[k % NBUF].at[:, pl.ds(0, CHUNK_SIZES[k])],
            isems[k % NBUF],
        )

    def out_copy(k):
        return pltpu.make_async_copy(
            obufs[k % 2].at[pl.ds(0, CHUNK_SIZES[k])],
            out_hbm.at[pl.ds(base + CHUNK_OFFS[k], CHUNK_SIZES[k])],
            osems[k % 2],
        )

    def reduce_cols(src, dst, n_cols):
        @plsc.parallel_loop(0, n_cols // LANES, unroll=1)
        def col_group(j):
            off = j * LANES
            vals = [src[i, pl.ds(off, LANES)] for i in range(N_VOTERS)]
            while len(vals) > 1:
                vals = [
                    vals[i] + vals[i + 1] for i in range(0, len(vals), 2)
                ]
            dst[pl.ds(off, LANES)] = jnp.where(
                vals[0] > N_VOTERS // 2, 1, 0
            ).astype(jnp.int32)

    for k in range(NBUF):
        in_copy(k).start()
    for k in range(NCHUNKS):
        in_copy(k).wait()
        if k >= 2:
            out_copy(k - 2).wait()
        reduce_cols(bufs[k % NBUF], obufs[k % 2], CHUNK_SIZES[k])
        out_copy(k).start()
        if k + NBUF < NCHUNKS:
            in_copy(k + NBUF).start()
    out_copy(NCHUNKS - 2).wait()
    out_copy(NCHUNKS - 1).wait()


def _tc_body(x_ref, o_ref):
    o_ref[...] = (
        jnp.sum(x_ref[...], axis=0) > N_VOTERS // 2
    ).astype(jnp.int32)


@jax.jit
def _vote(inputs):
    out_tc = pl.pallas_call(
        _tc_body,
        grid=(TC_BLOCKS,),
        in_specs=[
            pl.BlockSpec(
                (N_VOTERS, TC_BLOCK), lambda j: (0, j + SC_COLS // TC_BLOCK)
            )
        ],
        out_specs=pl.BlockSpec((TC_BLOCK,), lambda j: (j,)),
        out_shape=jax.ShapeDtypeStruct((TC_COLS,), jnp.int32),
    )(inputs)
    k = pl.kernel(
        _body,
        out_type=jax.ShapeDtypeStruct((SC_COLS,), jnp.int32),
        mesh=plsc.VectorSubcoreMesh(core_axis_name="c", subcore_axis_name="s"),
        scratch_types=[
            pltpu.VMEM((N_VOTERS, CHUNK), jnp.int32),
            pltpu.VMEM((N_VOTERS, CHUNK), jnp.int32),
            pltpu.VMEM((N_VOTERS, CHUNK), jnp.int32),
            pltpu.VMEM((CHUNK,), jnp.int32),
            pltpu.VMEM((CHUNK,), jnp.int32),
            pltpu.SemaphoreType.DMA,
            pltpu.SemaphoreType.DMA,
            pltpu.SemaphoreType.DMA,
            pltpu.SemaphoreType.DMA,
            pltpu.SemaphoreType.DMA,
        ],
    )
    out_sc = k(inputs)
    return jnp.concatenate([out_sc, out_tc])


def kernel(inputs):
    return _vote(inputs)
